# baseline (speedup 1.0000x reference)
import functools

import jax
import jax.numpy as jnp
import numpy as np
from jax import lax
from jax.experimental import pallas as pl
from jax.experimental.pallas import tpu as pltpu
from jax.experimental.pallas import tpu_sc as plsc

N = 10000
E = 320000
D = 128
DE = 16
H = 4

NC, NS, L = 2, 16, 16
NW = NC * NS
NP = 10240
EP = 327680
EW = EP // NW
K = 1024
CHUNKS = EW // K
GRP = 128
NR = NP // NS

BQ = 8192
BN = 2000


def _p_body(eft_ref, wa_ref, nf_ref, ei_ref, p_ref, g_ref, dst_ref):
    i = pl.program_id(0)
    wa_e = jnp.mean(wa_ref[D:, :], axis=1).reshape(1, DE)
    lanes = i * BQ + lax.broadcasted_iota(jnp.int32, (1, BQ), 1)
    mask = (lanes < E)[0]
    score = jnp.dot(wa_e, eft_ref[...],
                    preferred_element_type=jnp.float32)
    p_ref[...] = jnp.where(mask, jnp.exp(score[0]), 0.0)
    dst = jnp.where(lanes < E, ei_ref[1:2, :], 0)
    dst_ref[...] = dst.reshape(1, BQ // 128, 128).reshape(BQ // 128, 128)

    @pl.when(i == 0)
    def _():
        wa_n = jnp.mean(wa_ref[:D, :], axis=1, keepdims=True)
        col = jnp.exp(jnp.dot(nf_ref[...], wa_n,
                              preferred_element_type=jnp.float32))
        g_ref[...] = jnp.concatenate(
            [col.reshape(N), jnp.ones((NP - N,), jnp.float32)])


_tc_p = pl.pallas_call(
    _p_body,
    grid=(EP // BQ,),
    in_specs=[
        pl.BlockSpec((DE, BQ), lambda i: (0, i)),
        pl.BlockSpec((D + DE, H), lambda i: (0, 0)),
        pl.BlockSpec((N, D), lambda i: (0, 0)),
        pl.BlockSpec((2, BQ), lambda i: (0, i)),
    ],
    out_specs=[
        pl.BlockSpec((BQ,), lambda i: (i,)),
        pl.BlockSpec((NP,), lambda i: (0,)),
        pl.BlockSpec((BQ // 128, 128), lambda i: (i, 0)),
    ],
    out_shape=[
        jax.ShapeDtypeStruct((EP,), jnp.float32),
        jax.ShapeDtypeStruct((NP,), jnp.float32),
        jax.ShapeDtypeStruct((EP // 128, 128), jnp.int32),
    ],
)


TCC = K // 128
TCE = E // 128


def _sc_body(dst_hbm, p_hbm, ef4_hbm, g_hbm, u_out, s_out,
             g_v, dst_v, p_v, v4, ef_v, w_v, u_sh, s_sh,
             sem_i0, sem_i1, sem_s0, sem_s1):
    c = lax.axis_index("c")
    s = lax.axis_index("s")
    wid = c * NS + s
    sem_in = [sem_i0, sem_i1]
    sem_sc = [sem_s0, sem_s1]

    g_d = pltpu.async_copy(g_hbm, g_v, sem_i0)

    def _zu(i, x):
        ef_v[0, i, :] = jnp.zeros((L,), jnp.float32)
        return x

    lax.fori_loop(0, NR, _zu, 0)

    def _zw(i, x):
        w_v[0, pl.ds(i * L, L)] = jnp.zeros((L,), jnp.float32)
        return x

    lax.fori_loop(0, NR // L, _zw, 0)
    pltpu.sync_copy(ef_v.at[0, pl.ds(0, NR)], u_sh.at[pl.ds(s * NR, NR)])
    pltpu.sync_copy(w_v.at[0, pl.ds(0, NR)], s_sh.at[pl.ds(s * NR, NR)])
    g_d.wait()
    plsc.subcore_barrier()

    def _ef_in(ci, issue):
        b = ci % 2
        tcb = wid * (EW // 128) + ci * TCC
        valid = EW - (EP - E)

        def _go(src, dst):
            if issue:
                pltpu.async_copy(src, dst, sem_in[b])
            else:
                pltpu.make_async_copy(src, dst, sem_in[b]).wait()

        if ci * K + K <= valid:
            _go(ef4_hbm.at[:, pl.ds(tcb, TCC)], v4.at[b])
        else:
            @pl.when(wid != NW - 1)
            def _():
                _go(ef4_hbm.at[:, pl.ds(tcb, TCC)], v4.at[b])

            if ci * K < valid:
                nv = (valid - ci * K) // 128

                @pl.when(wid == NW - 1)
                def _():
                    _go(ef4_hbm.at[:, pl.ds(TCE - nv, nv)],
                        v4.at[b, :, pl.ds(0, nv)])

    def start_in(ci):
        b = ci % 2
        ebase = wid * EW + ci * K
        rbase = wid * (EW // GRP) + ci * (K // GRP)
        ds_ = [
            pltpu.async_copy(dst_hbm.at[pl.ds(rbase, K // GRP)],
                             dst_v.at[ci % 3], sem_in[b]),
            pltpu.async_copy(p_hbm.at[pl.ds(ebase, K)],
                             p_v.at[b], sem_in[b]),
        ]
        _ef_in(ci, issue=True)
        return ds_

    def compute(ci):
        b = ci % 2
        db = ci % 3
        iota = lax.iota(jnp.int32, L)

        @plsc.parallel_loop(0, K // L, unroll=4)
        def _grp16(j):
            r = j // (GRP // L)
            off = (j % (GRP // L)) * L
            idx = dst_v[db, r, pl.ds(off, L)]
            gv = plsc.load_gather(g_v, [idx])
            w16 = gv * p_v[b, pl.ds(j * L, L)]
            w_v[b, pl.ds(j * L, L)] = w16
            rows = j * L + iota
            for k in range(DE):
                vals = v4[b, k // 8, r, k % 8, pl.ds(off, L)] * w16
                plsc.store_scatter(
                    ef_v.at[b], [rows, jnp.full((L,), k, jnp.int32)], vals)

    def fire_scatter(ci):
        b = ci % 2
        db = ci % 3
        ds_ = []
        for grp in range(K // GRP):
            ds_.append(pltpu.async_copy(
                ef_v.at[b, pl.ds(grp * GRP, GRP)],
                u_sh.at[dst_v.at[db, grp]], sem_sc[b], add=True))
            ds_.append(pltpu.async_copy(
                w_v.at[b, pl.ds(grp * GRP, GRP)],
                s_sh.at[dst_v.at[db, grp]], sem_sc[b], add=True))
        return ds_

    in_d = {}
    sc_d = {}
    in_d[0] = start_in(0)
    for ci in range(CHUNKS):
        if ci >= 2:
            for d in sc_d.pop(ci - 2):
                d.wait()
        if ci + 1 < CHUNKS:
            in_d[ci + 1] = start_in(ci + 1)
        for d in in_d.pop(ci):
            d.wait()
        _ef_in(ci, issue=False)
        compute(ci)
        sc_d[ci] = fire_scatter(ci)
    for ci in (CHUNKS - 2, CHUNKS - 1):
        for d in sc_d.pop(ci):
            d.wait()

    plsc.subcore_barrier()

    pltpu.sync_copy(u_sh.at[pl.ds(s * NR, NR)], ef_v.at[0, pl.ds(0, NR)])
    pltpu.sync_copy(ef_v.at[0, pl.ds(0, NR)],
                    u_out.at[pl.ds(c * NP + s * NR, NR)])
    pltpu.sync_copy(s_sh.at[pl.ds(s * NR, NR)], w_v.at[0, pl.ds(0, NR)])
    pltpu.sync_copy(w_v.at[0, pl.ds(0, NR)],
                    s_out.at[pl.ds(c * NP + s * NR, NR)])


_sc_call = pl.kernel(
    _sc_body,
    out_type=[
        jax.ShapeDtypeStruct((NC * NP, DE), jnp.float32),
        jax.ShapeDtypeStruct((NC * NP,), jnp.float32),
    ],
    mesh=plsc.VectorSubcoreMesh(core_axis_name="c", subcore_axis_name="s"),
    compiler_params=pltpu.CompilerParams(
        needs_layout_passes=False, use_tc_tiling_on_sc=False),
    scratch_types=[
        pltpu.VMEM((NP,), jnp.float32),
        pltpu.VMEM((3, K // GRP, GRP), jnp.int32),
        pltpu.VMEM((2, K), jnp.float32),
        pltpu.VMEM((2, 2, TCC, 8, 128), jnp.float32),
        pltpu.VMEM((2, K, DE), jnp.float32),
        pltpu.VMEM((2, K), jnp.float32),
        pltpu.VMEM_SHARED((NP, DE), jnp.float32),
        pltpu.VMEM_SHARED((NP,), jnp.float32),
        pltpu.SemaphoreType.DMA,
        pltpu.SemaphoreType.DMA,
        pltpu.SemaphoreType.DMA,
        pltpu.SemaphoreType.DMA,
    ],
)


BN = 2048
UPB = BN * DE // 128
SPB = BN // 128

_R_NP = np.zeros((128, 16 * 128), np.float32)
for _m in range(16 * 128):
    _R_NP[8 * (_m // 128) + (_m % 128) // 16, _m] = 1.0


def _post_body(nf_ref, u0_ref, u1_ref, s0_ref, s1_ref,
               r_ref, wb_ref, wsbv_ref, w1_ref, b1_ref, w2_ref, b2_ref,
               o_ref):
    upk = u0_ref[...] + u1_ref[...]
    spk = s0_ref[...] + s1_ref[...]
    dp = jnp.dot(spk, r_ref[...], preferred_element_type=jnp.float32)
    dp = dp.reshape(SPB, 16, 128).reshape(UPB, 128)
    den = jnp.where(dp > 0, dp, 1.0)
    us = upk / den
    msg = jnp.dot(us, wb_ref[...], preferred_element_type=jnp.float32)
    msg = msg.reshape(UPB, 8, D).reshape(BN, D)
    rat = dp / den
    bvt = jnp.dot(rat, wsbv_ref[...], preferred_element_type=jnp.float32)
    msg = msg + bvt.reshape(UPB, 8, D).reshape(BN, D)
    h = jnp.maximum(
        jnp.dot(msg, w1_ref[...], preferred_element_type=jnp.float32)
        + b1_ref[...], 0.0)
    o_ref[...] = (nf_ref[...]
                  + jnp.dot(h, w2_ref[...], preferred_element_type=jnp.float32)
                  + b2_ref[...])


_tc_post = pl.pallas_call(
    _post_body,
    grid=(NP // BN,),
    in_specs=[
        pl.BlockSpec((BN, D), lambda i: (i, 0)),
        pl.BlockSpec((UPB, 128), lambda i: (i, 0)),
        pl.BlockSpec((UPB, 128), lambda i: (NP * DE // 128 // UPB + i, 0)),
        pl.BlockSpec((SPB, 128), lambda i: (i, 0)),
        pl.BlockSpec((SPB, 128), lambda i: (NP // 128 // SPB + i, 0)),
        pl.BlockSpec((128, 16 * 128), lambda i: (0, 0)),
        pl.BlockSpec((128, 8 * D), lambda i: (0, 0)),
        pl.BlockSpec((128, 8 * D), lambda i: (0, 0)),
        pl.BlockSpec((D, D), lambda i: (0, 0)),
        pl.BlockSpec((1, D), lambda i: (0, 0)),
        pl.BlockSpec((D, D), lambda i: (0, 0)),
        pl.BlockSpec((1, D), lambda i: (0, 0)),
    ],
    out_specs=pl.BlockSpec((BN, D), lambda i: (i, 0)),
    out_shape=jax.ShapeDtypeStruct((N, D), jnp.float32),
)


def kernel(node_features, edge_features, edge_index, Wa, ba, Wv, bv,
           W1, b1, W2, b2):
    eft = edge_features.T
    ef4 = eft.reshape(2, 8, TCE, 128).transpose(0, 2, 1, 3)

    p, g_pad, dst2d = _tc_p(eft, Wa, node_features, edge_index)

    u_cat, s_cat = _sc_call(dst2d, p, ef4, g_pad)

    u_pack = u_cat.reshape(NC * NP * DE // 128, 128)
    s_pack = s_cat.reshape(NC * NP // 128, 128)
    wb = jnp.kron(jnp.eye(8, dtype=jnp.float32), Wv)
    wsbv = jnp.kron(jnp.eye(8, dtype=jnp.float32),
                    jnp.ones((DE, 1), jnp.float32) @ bv[None, :] / DE)
    return _tc_post(node_features, u_pack, u_pack, s_pack, s_pack,
                    jnp.asarray(_R_NP), wb, wsbv,
                    W1, b1.reshape(1, D), W2, b2.reshape(1, D))


# speedup vs baseline: 1.1042x; 1.1042x over previous
import functools

import jax
import jax.numpy as jnp
import numpy as np
from jax import lax
from jax.experimental import pallas as pl
from jax.experimental.pallas import tpu as pltpu
from jax.experimental.pallas import tpu_sc as plsc

N = 10000
E = 320000
D = 128
DE = 16
H = 4

NC, NS, L = 2, 16, 16
NW = NC * NS
NP = 10240
EP = 327680
EW = EP // NW
K = 1024
CHUNKS = EW // K
GRP = 128
NR = NP // NS

BQ = 8192
BN = 2000


def _p_body(eft_ref, wa_ref, nf_ref, ei_ref, p_ref, g_ref, dst_ref):
    i = pl.program_id(0)
    wa_e = jnp.mean(wa_ref[D:, :], axis=1).reshape(1, DE)
    lanes = i * BQ + lax.broadcasted_iota(jnp.int32, (1, BQ), 1)
    mask = (lanes < E)[0]
    score = jnp.dot(wa_e, eft_ref[...],
                    preferred_element_type=jnp.float32)
    p_ref[...] = jnp.where(mask, jnp.exp(score[0]), 0.0)
    dst = jnp.where(lanes < E, ei_ref[1:2, :], 0)
    dst_ref[...] = dst.reshape(1, BQ // 128, 128).reshape(BQ // 128, 128)

    @pl.when(i == 0)
    def _():
        wa_n = jnp.mean(wa_ref[:D, :], axis=1, keepdims=True)
        col = jnp.exp(jnp.dot(nf_ref[...], wa_n,
                              preferred_element_type=jnp.float32))
        g_ref[...] = jnp.concatenate(
            [col.reshape(N), jnp.ones((NP - N,), jnp.float32)])


_tc_p = pl.pallas_call(
    _p_body,
    grid=(EP // BQ,),
    in_specs=[
        pl.BlockSpec((DE, BQ), lambda i: (0, i)),
        pl.BlockSpec((D + DE, H), lambda i: (0, 0)),
        pl.BlockSpec((N, D), lambda i: (0, 0)),
        pl.BlockSpec((2, BQ), lambda i: (0, i)),
    ],
    out_specs=[
        pl.BlockSpec((BQ,), lambda i: (i,)),
        pl.BlockSpec((NP,), lambda i: (0,)),
        pl.BlockSpec((BQ // 128, 128), lambda i: (i, 0)),
    ],
    out_shape=[
        jax.ShapeDtypeStruct((EP,), jnp.float32),
        jax.ShapeDtypeStruct((NP,), jnp.float32),
        jax.ShapeDtypeStruct((EP // 128, 128), jnp.int32),
    ],
)


TCC = K // 128
TCE = E // 128


def _sc_body(dst_hbm, p_hbm, ef4_hbm, g_hbm, u_out, s_out,
             g_v, dst_v, p_v, v4, ef_v, w_v, u_sh, s_sh,
             sem_i0, sem_i1, sem_s0, sem_s1):
    c = lax.axis_index("c")
    s = lax.axis_index("s")
    wid = c * NS + s
    sem_in = [sem_i0, sem_i1]
    sem_sc = [sem_s0, sem_s1]

    g_d = pltpu.async_copy(g_hbm, g_v, sem_i0)

    def _zu(i, x):
        ef_v[0, i, :] = jnp.zeros((L,), jnp.float32)
        return x

    lax.fori_loop(0, NR, _zu, 0)

    def _zw(i, x):
        w_v[0, pl.ds(i * L, L)] = jnp.zeros((L,), jnp.float32)
        return x

    lax.fori_loop(0, NR // L, _zw, 0)
    pltpu.sync_copy(ef_v.at[0, pl.ds(0, NR)], u_sh.at[pl.ds(s * NR, NR)])
    pltpu.sync_copy(w_v.at[0, pl.ds(0, NR)], s_sh.at[pl.ds(s * NR, NR)])
    g_d.wait()
    plsc.subcore_barrier()

    def _ef_in(ci, issue):
        b = ci % 2
        tcb = wid * (EW // 128) + ci * TCC
        valid = EW - (EP - E)

        def _go(src, dst):
            if issue:
                pltpu.async_copy(src, dst, sem_in[b])
            else:
                pltpu.make_async_copy(src, dst, sem_in[b]).wait()

        if ci * K + K <= valid:
            _go(ef4_hbm.at[:, pl.ds(tcb, TCC)], v4.at[b])
        else:
            @pl.when(wid != NW - 1)
            def _():
                _go(ef4_hbm.at[:, pl.ds(tcb, TCC)], v4.at[b])

            if ci * K < valid:
                nv = (valid - ci * K) // 128

                @pl.when(wid == NW - 1)
                def _():
                    _go(ef4_hbm.at[:, pl.ds(TCE - nv, nv)],
                        v4.at[b, :, pl.ds(0, nv)])

    def start_in(ci):
        b = ci % 2
        ebase = wid * EW + ci * K
        rbase = wid * (EW // GRP) + ci * (K // GRP)
        ds_ = [
            pltpu.async_copy(dst_hbm.at[pl.ds(rbase, K // GRP)],
                             dst_v.at[ci % 3], sem_in[b]),
            pltpu.async_copy(p_hbm.at[pl.ds(ebase, K)],
                             p_v.at[b], sem_in[b]),
        ]
        _ef_in(ci, issue=True)
        return ds_

    def compute(ci):
        b = ci % 2
        db = ci % 3
        iota = lax.iota(jnp.int32, L)

        @plsc.parallel_loop(0, K // L, unroll=2)
        def _grp16(j):
            r = j // (GRP // L)
            off = (j % (GRP // L)) * L
            idx = dst_v[db, r, pl.ds(off, L)]
            gv = plsc.load_gather(g_v, [idx])
            w16 = gv * p_v[b, pl.ds(j * L, L)]
            w_v[b, pl.ds(j * L, L)] = w16
            rows = j * L + iota
            for k in range(DE):
                vals = v4[b, k // 8, r, k % 8, pl.ds(off, L)] * w16
                plsc.store_scatter(
                    ef_v.at[b], [rows, jnp.full((L,), k, jnp.int32)], vals)

    def fire_scatter(ci):
        b = ci % 2
        db = ci % 3
        ds_ = []
        for grp in range(K // GRP):
            ds_.append(pltpu.async_copy(
                ef_v.at[b, pl.ds(grp * GRP, GRP)],
                u_sh.at[dst_v.at[db, grp]], sem_sc[b], add=True))
            ds_.append(pltpu.async_copy(
                w_v.at[b, pl.ds(grp * GRP, GRP)],
                s_sh.at[dst_v.at[db, grp]], sem_sc[b], add=True))
        return ds_

    in_d = {}
    sc_d = {}
    in_d[0] = start_in(0)
    for ci in range(CHUNKS):
        if ci >= 2:
            for d in sc_d.pop(ci - 2):
                d.wait()
        if ci + 1 < CHUNKS:
            in_d[ci + 1] = start_in(ci + 1)
        for d in in_d.pop(ci):
            d.wait()
        _ef_in(ci, issue=False)
        compute(ci)
        sc_d[ci] = fire_scatter(ci)
    for ci in (CHUNKS - 2, CHUNKS - 1):
        for d in sc_d.pop(ci):
            d.wait()

    plsc.subcore_barrier()

    pltpu.sync_copy(u_sh.at[pl.ds(s * NR, NR)], ef_v.at[0, pl.ds(0, NR)])
    pltpu.sync_copy(ef_v.at[0, pl.ds(0, NR)],
                    u_out.at[pl.ds(c * NP + s * NR, NR)])
    pltpu.sync_copy(s_sh.at[pl.ds(s * NR, NR)], w_v.at[0, pl.ds(0, NR)])
    pltpu.sync_copy(w_v.at[0, pl.ds(0, NR)],
                    s_out.at[pl.ds(c * NP + s * NR, NR)])


_sc_call = pl.kernel(
    _sc_body,
    out_type=[
        jax.ShapeDtypeStruct((NC * NP, DE), jnp.float32),
        jax.ShapeDtypeStruct((NC * NP,), jnp.float32),
    ],
    mesh=plsc.VectorSubcoreMesh(core_axis_name="c", subcore_axis_name="s"),
    compiler_params=pltpu.CompilerParams(
        needs_layout_passes=False, use_tc_tiling_on_sc=False),
    scratch_types=[
        pltpu.VMEM((NP,), jnp.float32),
        pltpu.VMEM((3, K // GRP, GRP), jnp.int32),
        pltpu.VMEM((2, K), jnp.float32),
        pltpu.VMEM((2, 2, TCC, 8, 128), jnp.float32),
        pltpu.VMEM((2, K, DE), jnp.float32),
        pltpu.VMEM((2, K), jnp.float32),
        pltpu.VMEM_SHARED((NP, DE), jnp.float32),
        pltpu.VMEM_SHARED((NP,), jnp.float32),
        pltpu.SemaphoreType.DMA,
        pltpu.SemaphoreType.DMA,
        pltpu.SemaphoreType.DMA,
        pltpu.SemaphoreType.DMA,
    ],
)


BN = 2048
UPB = BN * DE // 128
SPB = BN // 128

_R_NP = np.zeros((128, 16 * 128), np.float32)
for _m in range(16 * 128):
    _R_NP[8 * (_m // 128) + (_m % 128) // 16, _m] = 1.0


def _post_body(nf_ref, u0_ref, u1_ref, s0_ref, s1_ref,
               r_ref, wb_ref, wsbv_ref, w1_ref, b1_ref, w2_ref, b2_ref,
               o_ref):
    upk = u0_ref[...] + u1_ref[...]
    spk = s0_ref[...] + s1_ref[...]
    dp = jnp.dot(spk, r_ref[...], preferred_element_type=jnp.float32)
    dp = dp.reshape(SPB, 16, 128).reshape(UPB, 128)
    den = jnp.where(dp > 0, dp, 1.0)
    us = upk / den
    msg = jnp.dot(us, wb_ref[...], preferred_element_type=jnp.float32)
    msg = msg.reshape(UPB, 8, D).reshape(BN, D)
    rat = dp / den
    bvt = jnp.dot(rat, wsbv_ref[...], preferred_element_type=jnp.float32)
    msg = msg + bvt.reshape(UPB, 8, D).reshape(BN, D)
    h = jnp.maximum(
        jnp.dot(msg, w1_ref[...], preferred_element_type=jnp.float32)
        + b1_ref[...], 0.0)
    o_ref[...] = (nf_ref[...]
                  + jnp.dot(h, w2_ref[...], preferred_element_type=jnp.float32)
                  + b2_ref[...])


_tc_post = pl.pallas_call(
    _post_body,
    grid=(NP // BN,),
    in_specs=[
        pl.BlockSpec((BN, D), lambda i: (i, 0)),
        pl.BlockSpec((UPB, 128), lambda i: (i, 0)),
        pl.BlockSpec((UPB, 128), lambda i: (NP * DE // 128 // UPB + i, 0)),
        pl.BlockSpec((SPB, 128), lambda i: (i, 0)),
        pl.BlockSpec((SPB, 128), lambda i: (NP // 128 // SPB + i, 0)),
        pl.BlockSpec((128, 16 * 128), lambda i: (0, 0)),
        pl.BlockSpec((128, 8 * D), lambda i: (0, 0)),
        pl.BlockSpec((128, 8 * D), lambda i: (0, 0)),
        pl.BlockSpec((D, D), lambda i: (0, 0)),
        pl.BlockSpec((1, D), lambda i: (0, 0)),
        pl.BlockSpec((D, D), lambda i: (0, 0)),
        pl.BlockSpec((1, D), lambda i: (0, 0)),
    ],
    out_specs=pl.BlockSpec((BN, D), lambda i: (i, 0)),
    out_shape=jax.ShapeDtypeStruct((N, D), jnp.float32),
)


def kernel(node_features, edge_features, edge_index, Wa, ba, Wv, bv,
           W1, b1, W2, b2):
    eft = edge_features.T
    ef4 = eft.reshape(2, 8, TCE, 128).transpose(0, 2, 1, 3)

    p, g_pad, dst2d = _tc_p(eft, Wa, node_features, edge_index)

    u_cat, s_cat = _sc_call(dst2d, p, ef4, g_pad)

    u_pack = u_cat.reshape(NC * NP * DE // 128, 128)
    s_pack = s_cat.reshape(NC * NP // 128, 128)
    wb = jnp.kron(jnp.eye(8, dtype=jnp.float32), Wv)
    wsbv = jnp.kron(jnp.eye(8, dtype=jnp.float32),
                    jnp.ones((DE, 1), jnp.float32) @ bv[None, :] / DE)
    return _tc_post(node_features, u_pack, u_pack, s_pack, s_pack,
                    jnp.asarray(_R_NP), wb, wsbv,
                    W1, b1.reshape(1, D), W2, b2.reshape(1, D))


# speedup vs baseline: 1.3777x; 1.2476x over previous
import functools

import jax
import jax.numpy as jnp
import numpy as np
from jax import lax
from jax.experimental import pallas as pl
from jax.experimental.pallas import tpu as pltpu
from jax.experimental.pallas import tpu_sc as plsc

N = 10000
E = 320000
D = 128
DE = 16
H = 4

NC, NS, L = 2, 16, 16
NW = NC * NS
NP = 10240
EP = 327680
EW = EP // NW
K = 1024
CHUNKS = EW // K
GRP = 128
NR = NP // NS

BQ = 8192
BN = 2000


def _g_body(nf_ref, wa_ref, g_ref):
    wa_n = jnp.mean(wa_ref[:D, :], axis=1, keepdims=True)
    col = jnp.exp(jnp.dot(nf_ref[...], wa_n,
                          preferred_element_type=jnp.float32))
    g_ref[...] = jnp.concatenate(
        [col.reshape(N), jnp.ones((NP - N,), jnp.float32)])


_tc_g = pl.pallas_call(
    _g_body,
    out_shape=jax.ShapeDtypeStruct((NP,), jnp.float32),
)


TCC = K // 128
TCE = E // 128


def _sc_body(ei3_hbm, ef4_hbm, g_hbm, wae_hbm, u_out, s_out,
             g_v, wae_v, dst_v, di_v, v4, ef_v, w_v, u_sh, s_sh,
             sem_i0, sem_i1, sem_s0, sem_s1):
    c = lax.axis_index("c")
    s = lax.axis_index("s")
    wid = c * NS + s
    sem_in = [sem_i0, sem_i1]
    sem_sc = [sem_s0, sem_s1]

    g_d = pltpu.async_copy(g_hbm, g_v, sem_i0)
    wae_d = pltpu.async_copy(wae_hbm, wae_v, sem_i0)

    def _zu(i, x):
        ef_v[0, i, :] = jnp.zeros((L,), jnp.float32)
        return x

    lax.fori_loop(0, NR, _zu, 0)

    def _zw(i, x):
        w_v[0, pl.ds(i * L, L)] = jnp.zeros((L,), jnp.float32)
        return x

    lax.fori_loop(0, NR // L, _zw, 0)
    pltpu.sync_copy(ef_v.at[0, pl.ds(0, NR)], u_sh.at[pl.ds(s * NR, NR)])
    pltpu.sync_copy(w_v.at[0, pl.ds(0, NR)], s_sh.at[pl.ds(s * NR, NR)])
    g_d.wait()
    wae_d.wait()
    plsc.subcore_barrier()

    def _ef_in(ci, issue):
        b = ci % 2
        db = ci % 3
        tcb = wid * (EW // 128) + ci * TCC
        valid = EW - (EP - E)

        def _go(src, dst):
            if issue:
                pltpu.async_copy(src, dst, sem_in[b])
            else:
                pltpu.make_async_copy(src, dst, sem_in[b]).wait()

        def _both(nv):
            if nv == TCC:
                _go(ef4_hbm.at[:, pl.ds(tcb, TCC)], v4.at[b])
                _go(ei3_hbm.at[pl.ds(tcb, TCC)], dst_v.at[db])
            else:
                _go(ef4_hbm.at[:, pl.ds(TCE - nv, nv)],
                    v4.at[b, :, pl.ds(0, nv)])
                _go(ei3_hbm.at[pl.ds(TCE - nv, nv)],
                    dst_v.at[db, pl.ds(0, nv)])

        if ci * K + K <= valid:
            _both(TCC)
        else:
            @pl.when(wid != NW - 1)
            def _():
                _both(TCC)

            if ci * K < valid:
                nv = (valid - ci * K) // 128

                @pl.when(wid == NW - 1)
                def _():
                    _both(nv)

    def start_in(ci):
        _ef_in(ci, issue=True)
        return []

    def compute(ci):
        b = ci % 2
        db = ci % 3
        iota = lax.iota(jnp.int32, L)
        ebase = wid * EW + ci * K
        wae = wae_v[...]

        @plsc.parallel_loop(0, K // L, unroll=2)
        def _grp16(j):
            r = j // (GRP // L)
            off = (j % (GRP // L)) * L
            e16 = ebase + j * L + iota
            raw = dst_v[db, r, 1, pl.ds(off, L)]
            idx = jnp.where(e16 < E, raw, N)
            di_v[db, r, pl.ds(off, L)] = idx
            gv = plsc.load_gather(g_v, [idx])
            vs = []
            score = None
            for k in range(DE):
                vk = v4[b, k // 8, r, k % 8, pl.ds(off, L)]
                vs.append(vk)
                term = vk * wae[k]
                score = term if score is None else score + term
            w16 = gv * jnp.exp(score)
            w_v[b, pl.ds(j * L, L)] = w16
            rows = j * L + iota
            for k in range(DE):
                plsc.store_scatter(
                    ef_v.at[b], [rows, jnp.full((L,), k, jnp.int32)],
                    vs[k] * w16)

    def fire_scatter(ci):
        b = ci % 2
        db = ci % 3
        ds_ = []
        for grp in range(K // GRP):
            ds_.append(pltpu.async_copy(
                ef_v.at[b, pl.ds(grp * GRP, GRP)],
                u_sh.at[di_v.at[db, grp]], sem_sc[b], add=True))
            ds_.append(pltpu.async_copy(
                w_v.at[b, pl.ds(grp * GRP, GRP)],
                s_sh.at[di_v.at[db, grp]], sem_sc[b], add=True))
        return ds_

    in_d = {}
    sc_d = {}
    in_d[0] = start_in(0)
    for ci in range(CHUNKS):
        if ci >= 2:
            for d in sc_d.pop(ci - 2):
                d.wait()
        if ci + 1 < CHUNKS:
            in_d[ci + 1] = start_in(ci + 1)
        for d in in_d.pop(ci):
            d.wait()
        _ef_in(ci, issue=False)
        compute(ci)
        sc_d[ci] = fire_scatter(ci)
    for ci in (CHUNKS - 2, CHUNKS - 1):
        for d in sc_d.pop(ci):
            d.wait()

    plsc.subcore_barrier()

    pltpu.sync_copy(u_sh.at[pl.ds(s * NR, NR)], ef_v.at[0, pl.ds(0, NR)])
    pltpu.sync_copy(ef_v.at[0, pl.ds(0, NR)],
                    u_out.at[pl.ds(c * NP + s * NR, NR)])
    pltpu.sync_copy(s_sh.at[pl.ds(s * NR, NR)], w_v.at[0, pl.ds(0, NR)])
    pltpu.sync_copy(w_v.at[0, pl.ds(0, NR)],
                    s_out.at[pl.ds(c * NP + s * NR, NR)])


_sc_call = pl.kernel(
    _sc_body,
    out_type=[
        jax.ShapeDtypeStruct((NC * NP, DE), jnp.float32),
        jax.ShapeDtypeStruct((NC * NP,), jnp.float32),
    ],
    mesh=plsc.VectorSubcoreMesh(core_axis_name="c", subcore_axis_name="s"),
    compiler_params=pltpu.CompilerParams(
        needs_layout_passes=False, use_tc_tiling_on_sc=False),
    scratch_types=[
        pltpu.VMEM((NP,), jnp.float32),
        pltpu.VMEM((DE,), jnp.float32),
        pltpu.VMEM((3, TCC, 2, 128), jnp.int32),
        pltpu.VMEM((3, TCC, 128), jnp.int32),
        pltpu.VMEM((2, 2, TCC, 8, 128), jnp.float32),
        pltpu.VMEM((2, K, DE), jnp.float32),
        pltpu.VMEM((2, K), jnp.float32),
        pltpu.VMEM_SHARED((NP, DE), jnp.float32),
        pltpu.VMEM_SHARED((NP,), jnp.float32),
        pltpu.SemaphoreType.DMA,
        pltpu.SemaphoreType.DMA,
        pltpu.SemaphoreType.DMA,
        pltpu.SemaphoreType.DMA,
    ],
)


BN = 2048
UPB = BN * DE // 128
SPB = BN // 128

_R_NP = np.zeros((128, 16 * 128), np.float32)
for _m in range(16 * 128):
    _R_NP[8 * (_m // 128) + (_m % 128) // 16, _m] = 1.0


def _post_body(nf_ref, u0_ref, u1_ref, s0_ref, s1_ref,
               r_ref, wb_ref, wsbv_ref, w1_ref, b1_ref, w2_ref, b2_ref,
               o_ref):
    upk = u0_ref[...] + u1_ref[...]
    spk = s0_ref[...] + s1_ref[...]
    dp = jnp.dot(spk, r_ref[...], preferred_element_type=jnp.float32)
    dp = dp.reshape(SPB, 16, 128).reshape(UPB, 128)
    den = jnp.where(dp > 0, dp, 1.0)
    us = upk / den
    msg = jnp.dot(us, wb_ref[...], preferred_element_type=jnp.float32)
    msg = msg.reshape(UPB, 8, D).reshape(BN, D)
    rat = dp / den
    bvt = jnp.dot(rat, wsbv_ref[...], preferred_element_type=jnp.float32)
    msg = msg + bvt.reshape(UPB, 8, D).reshape(BN, D)
    h = jnp.maximum(
        jnp.dot(msg, w1_ref[...], preferred_element_type=jnp.float32)
        + b1_ref[...], 0.0)
    o_ref[...] = (nf_ref[...]
                  + jnp.dot(h, w2_ref[...], preferred_element_type=jnp.float32)
                  + b2_ref[...])


_tc_post = pl.pallas_call(
    _post_body,
    grid=(NP // BN,),
    in_specs=[
        pl.BlockSpec((BN, D), lambda i: (i, 0)),
        pl.BlockSpec((UPB, 128), lambda i: (i, 0)),
        pl.BlockSpec((UPB, 128), lambda i: (NP * DE // 128 // UPB + i, 0)),
        pl.BlockSpec((SPB, 128), lambda i: (i, 0)),
        pl.BlockSpec((SPB, 128), lambda i: (NP // 128 // SPB + i, 0)),
        pl.BlockSpec((128, 16 * 128), lambda i: (0, 0)),
        pl.BlockSpec((128, 8 * D), lambda i: (0, 0)),
        pl.BlockSpec((128, 8 * D), lambda i: (0, 0)),
        pl.BlockSpec((D, D), lambda i: (0, 0)),
        pl.BlockSpec((1, D), lambda i: (0, 0)),
        pl.BlockSpec((D, D), lambda i: (0, 0)),
        pl.BlockSpec((1, D), lambda i: (0, 0)),
    ],
    out_specs=pl.BlockSpec((BN, D), lambda i: (i, 0)),
    out_shape=jax.ShapeDtypeStruct((N, D), jnp.float32),
)


def kernel(node_features, edge_features, edge_index, Wa, ba, Wv, bv,
           W1, b1, W2, b2):
    eft = edge_features.T
    ef4 = eft.reshape(2, 8, TCE, 128).transpose(0, 2, 1, 3)
    ei3 = edge_index.reshape(2, TCE, 128).transpose(1, 0, 2)
    wae = jnp.mean(Wa[D:, :], axis=1)

    g_pad = _tc_g(node_features, Wa)

    u_cat, s_cat = _sc_call(ei3, ef4, g_pad, wae)

    u_pack = u_cat.reshape(NC * NP * DE // 128, 128)
    s_pack = s_cat.reshape(NC * NP // 128, 128)
    wb = jnp.kron(jnp.eye(8, dtype=jnp.float32), Wv)
    wsbv = jnp.kron(jnp.eye(8, dtype=jnp.float32),
                    jnp.ones((DE, 1), jnp.float32) @ bv[None, :] / DE)
    return _tc_post(node_features, u_pack, u_pack, s_pack, s_pack,
                    jnp.asarray(_R_NP), wb, wsbv,
                    W1, b1.reshape(1, D), W2, b2.reshape(1, D))


# speedup vs baseline: 1.6028x; 1.1634x over previous
import functools

import jax
import jax.numpy as jnp
import numpy as np
from jax import lax
from jax.experimental import pallas as pl
from jax.experimental.pallas import tpu as pltpu
from jax.experimental.pallas import tpu_sc as plsc

N = 10000
E = 320000
D = 128
DE = 16
H = 4

NC, NS, L = 2, 16, 16
NW = NC * NS
NP = 10240
EP = 327680
EW = EP // NW
K = 1024
CHUNKS = EW // K
GRP = 128
NR = NP // NS

BQ = 8192
BN = 2000


def _g_body(nf_ref, wa_ref, g_ref):
    wa_n = jnp.mean(wa_ref[:D, :], axis=1, keepdims=True)
    col = jnp.exp(jnp.dot(nf_ref[...], wa_n,
                          preferred_element_type=jnp.float32))
    g_ref[...] = jnp.concatenate(
        [col.reshape(N), jnp.ones((NP - N,), jnp.float32)])


_tc_g = pl.pallas_call(
    _g_body,
    out_shape=jax.ShapeDtypeStruct((NP,), jnp.float32),
)


TCC = K // 128
TCE = E // 128


def _sc_body(ei3_hbm, ef4_hbm, g_hbm, wae_hbm, u_out, s_out,
             g_v, wae_v, dst_v, di_v, v4, ef_v, w_v, u_sh, s_sh,
             sem_i0, sem_i1, sem_s0, sem_s1):
    c = lax.axis_index("c")
    s = lax.axis_index("s")
    wid = c * NS + s
    sem_in = [sem_i0, sem_i1]
    sem_sc = [sem_s0, sem_s1]

    g_d = pltpu.async_copy(g_hbm, g_v, sem_i0)
    wae_d = pltpu.async_copy(wae_hbm, wae_v, sem_i0)

    def _zu(i, x):
        ef_v[0, i, :] = jnp.zeros((L,), jnp.float32)
        return x

    lax.fori_loop(0, NR, _zu, 0)

    def _zw(i, x):
        w_v[0, pl.ds(i * L, L)] = jnp.zeros((L,), jnp.float32)
        return x

    lax.fori_loop(0, NR // L, _zw, 0)
    pltpu.sync_copy(ef_v.at[0, pl.ds(0, NR)], u_sh.at[pl.ds(s * NR, NR)])
    pltpu.sync_copy(w_v.at[0, pl.ds(0, NR)], s_sh.at[pl.ds(s * NR, NR)])
    g_d.wait()
    wae_d.wait()
    plsc.subcore_barrier()

    def _ef_in(ci, issue):
        b = ci % 2
        db = ci % 3
        tcb = wid * (EW // 128) + ci * TCC
        valid = EW - (EP - E)

        def _go(src, dst):
            if issue:
                pltpu.async_copy(src, dst, sem_in[b])
            else:
                pltpu.make_async_copy(src, dst, sem_in[b]).wait()

        def _both(nv):
            if nv == TCC:
                _go(ef4_hbm.at[:, pl.ds(tcb, TCC)], v4.at[b])
                _go(ei3_hbm.at[pl.ds(tcb, TCC)], dst_v.at[db])
            else:
                _go(ef4_hbm.at[:, pl.ds(TCE - nv, nv)],
                    v4.at[b, :, pl.ds(0, nv)])
                _go(ei3_hbm.at[pl.ds(TCE - nv, nv)],
                    dst_v.at[db, pl.ds(0, nv)])

        if ci * K + K <= valid:
            _both(TCC)
        else:
            @pl.when(wid != NW - 1)
            def _():
                _both(TCC)

            if ci * K < valid:
                nv = (valid - ci * K) // 128

                @pl.when(wid == NW - 1)
                def _():
                    _both(nv)

    def start_in(ci):
        _ef_in(ci, issue=True)
        return []

    def compute(ci):
        b = ci % 2
        db = ci % 3
        iota = lax.iota(jnp.int32, L)
        ebase = wid * EW + ci * K
        wae = wae_v[...]

        @plsc.parallel_loop(0, K // L, unroll=1)
        def _grp16(j):
            r = j // (GRP // L)
            off = (j % (GRP // L)) * L
            e16 = ebase + j * L + iota
            raw = dst_v[db, r, 1, pl.ds(off, L)]
            idx = jnp.where(e16 < E, raw, N)
            di_v[db, r, pl.ds(off, L)] = idx
            gv = plsc.load_gather(g_v, [idx])
            vs = []
            score = None
            for k in range(DE):
                vk = v4[b, k // 8, r, k % 8, pl.ds(off, L)]
                vs.append(vk)
                term = vk * wae[k]
                score = term if score is None else score + term
            w16 = gv * jnp.exp(score)
            w_v[b, pl.ds(j * L, L)] = w16
            rows = j * L + iota
            for k in range(DE):
                plsc.store_scatter(
                    ef_v.at[b], [rows, jnp.full((L,), k, jnp.int32)],
                    vs[k] * w16)

    def fire_scatter(ci):
        b = ci % 2
        db = ci % 3
        ds_ = []
        for grp in range(K // GRP):
            ds_.append(pltpu.async_copy(
                ef_v.at[b, pl.ds(grp * GRP, GRP)],
                u_sh.at[di_v.at[db, grp]], sem_sc[b], add=True))
            ds_.append(pltpu.async_copy(
                w_v.at[b, pl.ds(grp * GRP, GRP)],
                s_sh.at[di_v.at[db, grp]], sem_sc[b], add=True))
        return ds_

    in_d = {}
    sc_d = {}
    in_d[0] = start_in(0)
    for ci in range(CHUNKS):
        if ci >= 2:
            for d in sc_d.pop(ci - 2):
                d.wait()
        if ci + 1 < CHUNKS:
            in_d[ci + 1] = start_in(ci + 1)
        for d in in_d.pop(ci):
            d.wait()
        _ef_in(ci, issue=False)
        compute(ci)
        sc_d[ci] = fire_scatter(ci)
    for ci in (CHUNKS - 2, CHUNKS - 1):
        for d in sc_d.pop(ci):
            d.wait()

    plsc.subcore_barrier()

    pltpu.sync_copy(u_sh.at[pl.ds(s * NR, NR)], ef_v.at[0, pl.ds(0, NR)])
    pltpu.sync_copy(ef_v.at[0, pl.ds(0, NR)],
                    u_out.at[pl.ds(c * NP + s * NR, NR)])
    pltpu.sync_copy(s_sh.at[pl.ds(s * NR, NR)], w_v.at[0, pl.ds(0, NR)])
    pltpu.sync_copy(w_v.at[0, pl.ds(0, NR)],
                    s_out.at[pl.ds(c * NP + s * NR, NR)])


_sc_call = pl.kernel(
    _sc_body,
    out_type=[
        jax.ShapeDtypeStruct((NC * NP, DE), jnp.float32),
        jax.ShapeDtypeStruct((NC * NP,), jnp.float32),
    ],
    mesh=plsc.VectorSubcoreMesh(core_axis_name="c", subcore_axis_name="s"),
    compiler_params=pltpu.CompilerParams(
        needs_layout_passes=False, use_tc_tiling_on_sc=False),
    scratch_types=[
        pltpu.VMEM((NP,), jnp.float32),
        pltpu.VMEM((DE,), jnp.float32),
        pltpu.VMEM((3, TCC, 2, 128), jnp.int32),
        pltpu.VMEM((3, TCC, 128), jnp.int32),
        pltpu.VMEM((2, 2, TCC, 8, 128), jnp.float32),
        pltpu.VMEM((2, K, DE), jnp.float32),
        pltpu.VMEM((2, K), jnp.float32),
        pltpu.VMEM_SHARED((NP, DE), jnp.float32),
        pltpu.VMEM_SHARED((NP,), jnp.float32),
        pltpu.SemaphoreType.DMA,
        pltpu.SemaphoreType.DMA,
        pltpu.SemaphoreType.DMA,
        pltpu.SemaphoreType.DMA,
    ],
)


BN = 2048
UPB = BN * DE // 128
SPB = BN // 128

_R_NP = np.zeros((128, 16 * 128), np.float32)
for _m in range(16 * 128):
    _R_NP[8 * (_m // 128) + (_m % 128) // 16, _m] = 1.0


def _post_body(nf_ref, u0_ref, u1_ref, s0_ref, s1_ref,
               r_ref, wb_ref, wsbv_ref, w1_ref, b1_ref, w2_ref, b2_ref,
               o_ref):
    upk = u0_ref[...] + u1_ref[...]
    spk = s0_ref[...] + s1_ref[...]
    dp = jnp.dot(spk, r_ref[...], preferred_element_type=jnp.float32)
    dp = dp.reshape(SPB, 16, 128).reshape(UPB, 128)
    den = jnp.where(dp > 0, dp, 1.0)
    us = upk / den
    msg = jnp.dot(us, wb_ref[...], preferred_element_type=jnp.float32)
    msg = msg.reshape(UPB, 8, D).reshape(BN, D)
    rat = dp / den
    bvt = jnp.dot(rat, wsbv_ref[...], preferred_element_type=jnp.float32)
    msg = msg + bvt.reshape(UPB, 8, D).reshape(BN, D)
    h = jnp.maximum(
        jnp.dot(msg, w1_ref[...], preferred_element_type=jnp.float32)
        + b1_ref[...], 0.0)
    o_ref[...] = (nf_ref[...]
                  + jnp.dot(h, w2_ref[...], preferred_element_type=jnp.float32)
                  + b2_ref[...])


_tc_post = pl.pallas_call(
    _post_body,
    grid=(NP // BN,),
    in_specs=[
        pl.BlockSpec((BN, D), lambda i: (i, 0)),
        pl.BlockSpec((UPB, 128), lambda i: (i, 0)),
        pl.BlockSpec((UPB, 128), lambda i: (NP * DE // 128 // UPB + i, 0)),
        pl.BlockSpec((SPB, 128), lambda i: (i, 0)),
        pl.BlockSpec((SPB, 128), lambda i: (NP // 128 // SPB + i, 0)),
        pl.BlockSpec((128, 16 * 128), lambda i: (0, 0)),
        pl.BlockSpec((128, 8 * D), lambda i: (0, 0)),
        pl.BlockSpec((128, 8 * D), lambda i: (0, 0)),
        pl.BlockSpec((D, D), lambda i: (0, 0)),
        pl.BlockSpec((1, D), lambda i: (0, 0)),
        pl.BlockSpec((D, D), lambda i: (0, 0)),
        pl.BlockSpec((1, D), lambda i: (0, 0)),
    ],
    out_specs=pl.BlockSpec((BN, D), lambda i: (i, 0)),
    out_shape=jax.ShapeDtypeStruct((N, D), jnp.float32),
)


def kernel(node_features, edge_features, edge_index, Wa, ba, Wv, bv,
           W1, b1, W2, b2):
    eft = edge_features.T
    ef4 = eft.reshape(2, 8, TCE, 128).transpose(0, 2, 1, 3)
    ei3 = edge_index.reshape(2, TCE, 128).transpose(1, 0, 2)
    wae = jnp.mean(Wa[D:, :], axis=1)

    g_pad = _tc_g(node_features, Wa)

    u_cat, s_cat = _sc_call(ei3, ef4, g_pad, wae)

    u_pack = u_cat.reshape(NC * NP * DE // 128, 128)
    s_pack = s_cat.reshape(NC * NP // 128, 128)
    wb = jnp.kron(jnp.eye(8, dtype=jnp.float32), Wv)
    wsbv = jnp.kron(jnp.eye(8, dtype=jnp.float32),
                    jnp.ones((DE, 1), jnp.float32) @ bv[None, :] / DE)
    return _tc_post(node_features, u_pack, u_pack, s_pack, s_pack,
                    jnp.asarray(_R_NP), wb, wsbv,
                    W1, b1.reshape(1, D), W2, b2.reshape(1, D))
